# baseline (speedup 1.0000x reference)
import jax
import jax.numpy as jnp
from jax import lax
from jax.experimental import pallas as pl
from jax.experimental.pallas import tpu as pltpu
from jax.experimental.pallas import tpu_sc as plsc

NC = 2
NS = 16
NW = NC * NS
LANES = 16

B, C, H, W = 4, 96, 512, 512
P = B * H * W
PW = P // NW
CHUNK = 4096
NCHUNK = PW // CHUNK
HW = H * W
LOG2_HW = 18


GC = 1024
LOG2_GC = 10
CPAD = CHUNK + GC + 2 * LANES
TRASH = CPAD - 1


def _sc_body(z_hbm, l_hbm, lp_hbm, cond_hbm, out_hbm,
             l_v, lp_v, c_v, cig_v, cib_v, g_v, b_v, acc_v,
             sem_in0, sem_in1, sem_g0, sem_g1):
    wid = lax.axis_index("s") * NC + lax.axis_index("c")
    base = wid * PW
    iota = lax.iota(jnp.int32, LANES)
    zeros_i = jnp.zeros((LANES,), jnp.int32)
    sem_in = (sem_in0, sem_in1)
    sem_g = (sem_g0, sem_g1)

    def start_in(k):
        u = k % 2
        off = base + k * CHUNK
        return (
            pltpu.async_copy(l_hbm.at[pl.ds(off, CHUNK)], l_v[u], sem_in[u]),
            pltpu.async_copy(lp_hbm.at[pl.ds(off, CHUNK)], lp_v[u], sem_in[u]),
            pltpu.async_copy(cond_hbm.at[pl.ds(off, CHUNK)], c_v[u], sem_in[u]),
        )

    shift_idx = [jnp.maximum(iota - d, 0) for d in (1, 2, 4, 8)]
    shift_ok = [iota >= d for d in (1, 2, 4, 8)]
    zero_v = jnp.zeros((LANES,), jnp.int32)

    def do_compact(k):
        u = k % 2
        off = base + k * CHUNK

        def cbody(j, cnt):
            s = pl.ds(j * LANES, LANES)
            lv = l_v[u][s]
            lpv = lp_v[u][s]
            cv = c_v[u][s]
            p = (off + j * LANES) + iota
            bb = lax.shift_right_logical(p, LOG2_HW)
            pos = p + bb * ((C - 1) * HW)
            ig = pos + lax.shift_left(lv, LOG2_HW)
            ib = pos + lax.shift_left(lpv, LOG2_HW)
            m = (cv != 0) & (lv != lpv)
            v = m.astype(jnp.int32)
            for d in range(4):
                sh = v.at[shift_idx[d]].get(mode="promise_in_bounds")
                v = v + jnp.where(shift_ok[d], sh, 0)
            dest = jnp.where(m, (cnt - 1) + v, TRASH)
            plsc.store_scatter(cig_v[u], [dest], ig)
            plsc.store_scatter(cib_v[u], [dest], ib)
            return cnt + v[LANES - 1]

        nc = lax.fori_loop(0, CHUNK // LANES, cbody, 0, unroll=4)

        @plsc.parallel_loop(0, GC // LANES + 1, unroll=4)
        def _(t):
            s = pl.ds(nc + t * LANES, LANES)
            cig_v[u][s] = zero_v
            cib_v[u][s] = zero_v

        return nc

    def start_g(k, nc):
        u = k % 2
        ng = lax.shift_right_logical(nc + (GC - 1), LOG2_GC)

        def fire(s, c):
            sl = pl.ds(s * GC, GC)
            pltpu.async_copy(z_hbm.at[cig_v[u].at[sl]], g_v[u].at[sl],
                             sem_g[u])
            pltpu.async_copy(z_hbm.at[cib_v[u].at[sl]], b_v[u].at[sl],
                             sem_g[u])
            return c

        lax.fori_loop(0, ng, fire, 0)
        return ng

    def wait_g(k, ng):
        u = k % 2

        def drain(s, c):
            pltpu.make_async_copy(z_hbm.at[cig_v[u].at[pl.ds(0, GC)]],
                                  g_v[u].at[pl.ds(0, GC)], sem_g[u]).wait()
            pltpu.make_async_copy(z_hbm.at[cib_v[u].at[pl.ds(0, GC)]],
                                  b_v[u].at[pl.ds(0, GC)], sem_g[u]).wait()
            return c

        lax.fori_loop(0, ng, drain, 0)

    def do_acc(k, nc, acc):
        u = k % 2
        jt = lax.shift_right_logical(nc + 63, 6)

        def accbody(j, a):
            for t in range(4):
                s = pl.ds(j * 64 + t * LANES, LANES)
                a = a + (g_v[u][s] - b_v[u][s])
            return a

        return lax.fori_loop(0, jt, accbody, acc)

    acc = jnp.zeros((LANES,), jnp.float32)
    ins = [None] * (NCHUNK + 1)
    ncs = [None] * NCHUNK
    ngs = [None] * NCHUNK
    ins[0] = start_in(0)
    for d in ins[0]:
        d.wait()
    ncs[0] = do_compact(0)
    ins[1] = start_in(1)
    ngs[0] = start_g(0, ncs[0])
    for k in range(NCHUNK):
        if k + 1 < NCHUNK:
            for d in ins[k + 1]:
                d.wait()
            ncs[k + 1] = do_compact(k + 1)
            if k + 2 < NCHUNK:
                ins[k + 2] = start_in(k + 2)
        wait_g(k, ngs[k])
        if k + 1 < NCHUNK:
            ngs[k + 1] = start_g(k + 1, ncs[k + 1])
        acc = do_acc(k, ncs[k], acc)

    acc_v[...] = acc
    pltpu.sync_copy(acc_v, out_hbm.at[wid])


def _reduce_body(x_ref, o_ref):
    o_ref[0, 0] = jnp.sum(x_ref[...])


def _phys_view(x):
    s = x.shape[:-2]
    n = len(s)
    x6 = x.reshape(*s, H // 8, 8, W // 128, 128)
    return jnp.transpose(x6, tuple(range(n)) + (n, n + 2, n + 1, n + 3)
                         ).reshape(-1)


@jax.jit
def kernel(z, condition, l, l_prime):
    z_flat = _phys_view(z)
    l_i = _phys_view(l.astype(jnp.int32))
    lp_i = _phys_view(l_prime.astype(jnp.int32))
    c_i = _phys_view(condition.astype(jnp.int32))

    mesh = plsc.VectorSubcoreMesh(
        core_axis_name="c", subcore_axis_name="s",
        num_cores=NC, num_subcores=NS)
    partials = pl.kernel(
        _sc_body,
        out_type=jax.ShapeDtypeStruct((NW, LANES), jnp.float32),
        mesh=mesh,
        compiler_params=pltpu.CompilerParams(needs_layout_passes=False),
        scratch_types=[
            [pltpu.VMEM((CHUNK,), jnp.int32)] * 2,
            [pltpu.VMEM((CHUNK,), jnp.int32)] * 2,
            [pltpu.VMEM((CHUNK,), jnp.int32)] * 2,
            [pltpu.VMEM((CPAD,), jnp.int32)] * 2,
            [pltpu.VMEM((CPAD,), jnp.int32)] * 2,
            [pltpu.VMEM((CPAD,), jnp.float32)] * 2,
            [pltpu.VMEM((CPAD,), jnp.float32)] * 2,
            pltpu.VMEM((LANES,), jnp.float32),
            pltpu.SemaphoreType.DMA,
            pltpu.SemaphoreType.DMA,
            pltpu.SemaphoreType.DMA,
            pltpu.SemaphoreType.DMA,
        ],
    )(z_flat, l_i, lp_i, c_i)

    loss = pl.pallas_call(
        _reduce_body,
        out_shape=jax.ShapeDtypeStruct((1, 1), jnp.float32),
        out_specs=pl.BlockSpec(memory_space=pltpu.SMEM),
    )(partials.reshape(4, 128))
    return loss[0, 0]


# speedup vs baseline: 1.0197x; 1.0197x over previous
import jax
import jax.numpy as jnp
from jax import lax
from jax.experimental import pallas as pl
from jax.experimental.pallas import tpu as pltpu
from jax.experimental.pallas import tpu_sc as plsc

NC = 2
NS = 16
NW = NC * NS
LANES = 16

B, C, H, W = 4, 96, 512, 512
P = B * H * W
HW = H * W
LOG2_HW = 18

G0 = 12
Q0 = G0 * (P // 32)
CHUNK = 4096
PW = (P - Q0) // NW
NCHUNK = PW // CHUNK
HB = 64


def _sc_body(z_hbm, l_hbm, lp_hbm, cond_hbm, out_hbm,
             l_v, lp_v, c_v, ig_v, ib_v, g_v, b_v, acc_v,
             sem_in0, sem_in1, sem_g0, sem_g1):
    wid = lax.axis_index("s") * NC + lax.axis_index("c")
    base = Q0 + wid * PW
    iota = lax.iota(jnp.int32, LANES)
    sem_in = (sem_in0, sem_in1)
    sem_g = (sem_g0, sem_g1)

    def start_in(k):
        u = k % 2
        off = base + k * CHUNK
        return (
            pltpu.async_copy(l_hbm.at[pl.ds(off, CHUNK)], l_v[u], sem_in[u]),
            pltpu.async_copy(lp_hbm.at[pl.ds(off, CHUNK)], lp_v[u], sem_in[u]),
            pltpu.async_copy(cond_hbm.at[pl.ds(off, CHUNK)], c_v[u], sem_in[u]),
        )

    def do_idx(k):
        u = k % 2
        off = base + k * CHUNK

        @plsc.parallel_loop(0, CHUNK // LANES, unroll=8)
        def _(j):
            s = pl.ds(j * LANES, LANES)
            lv = l_v[u][s]
            lpv = lp_v[u][s]
            cv = c_v[u][s]
            p = (off + j * LANES) + iota
            bb = lax.shift_right_logical(p, LOG2_HW)
            pos = p + bb * ((C - 1) * HW)
            ig = pos + lax.shift_left(lv, LOG2_HW)
            ib = pos + lax.shift_left(lpv, LOG2_HW)
            ib = jnp.where(cv != 0, ib, ig)
            ig_v[u][s] = ig
            ib_v[u][s] = ib

    def start_g(k):
        u = k % 2
        return (
            pltpu.async_copy(z_hbm.at[ig_v[u]], g_v[u], sem_g[u]),
            pltpu.async_copy(z_hbm.at[ib_v[u]], b_v[u], sem_g[u]),
        )

    def do_acc(k, acc):
        u = k % 2

        @plsc.parallel_loop(0, CHUNK // LANES, unroll=8, carry=acc)
        def acc2(j, a):
            s = pl.ds(j * LANES, LANES)
            return a + (g_v[u][s] - b_v[u][s])

        return acc2

    acc = jnp.zeros((LANES,), jnp.float32)
    ins = [None] * (NCHUNK + 1)
    gs = [None] * (NCHUNK + 1)
    ins[0] = start_in(0)
    for d in ins[0]:
        d.wait()
    do_idx(0)
    ins[1] = start_in(1)
    gs[0] = start_g(0)
    for k in range(NCHUNK):
        if k + 1 < NCHUNK:
            for d in ins[k + 1]:
                d.wait()
            do_idx(k + 1)
            if k + 2 < NCHUNK:
                ins[k + 2] = start_in(k + 2)
        for d in gs[k]:
            d.wait()
        if k + 1 < NCHUNK:
            gs[k + 1] = start_g(k + 1)
        acc = do_acc(k, acc)

    acc_v[...] = acc
    pltpu.sync_copy(acc_v, out_hbm.at[wid])


def _tc_body(l_ref, lp_ref, c_ref, z_ref, o_ref, acc_ref, lm_ref, lpm_ref):
    g = pl.program_id(0)
    c = pl.program_id(1)

    @pl.when(jnp.logical_and(g == 0, c == 0))
    def _():
        acc_ref[...] = jnp.zeros_like(acc_ref)

    @pl.when(c == 0)
    def _():
        cb = c_ref[0]
        lm_ref[...] = jnp.where(cb != 0, l_ref[0], -1)
        lpm_ref[...] = jnp.where(cb != 0, lp_ref[0], -1)

    coeff = ((lm_ref[...] == c).astype(jnp.float32)
             - (lpm_ref[...] == c).astype(jnp.float32))
    acc_ref[...] += z_ref[0, 0] * coeff

    @pl.when(jnp.logical_and(g == G0 - 1, c == C - 1))
    def _():
        o_ref[0, 0] = jnp.sum(acc_ref[...])


def _reduce_body(x_ref, y_ref, o_ref):
    o_ref[0, 0] = jnp.sum(x_ref[...]) + y_ref[0, 0]


def _phys_view(x):
    s = x.shape[:-2]
    n = len(s)
    x6 = x.reshape(*s, H // 8, 8, W // 128, 128)
    return jnp.transpose(x6, tuple(range(n)) + (n, n + 2, n + 1, n + 3)
                         ).reshape(-1)


@jax.jit
def kernel(z, condition, l, l_prime):
    l32 = l.astype(jnp.int32)
    lp32 = l_prime.astype(jnp.int32)
    c32 = condition.astype(jnp.int32)

    mesh = plsc.VectorSubcoreMesh(
        core_axis_name="c", subcore_axis_name="s",
        num_cores=NC, num_subcores=NS)
    partials = pl.kernel(
        _sc_body,
        out_type=jax.ShapeDtypeStruct((NW, LANES), jnp.float32),
        mesh=mesh,
        compiler_params=pltpu.CompilerParams(needs_layout_passes=False),
        scratch_types=[
            [pltpu.VMEM((CHUNK,), jnp.int32)] * 2,
            [pltpu.VMEM((CHUNK,), jnp.int32)] * 2,
            [pltpu.VMEM((CHUNK,), jnp.int32)] * 2,
            [pltpu.VMEM((CHUNK,), jnp.int32)] * 2,
            [pltpu.VMEM((CHUNK,), jnp.int32)] * 2,
            [pltpu.VMEM((CHUNK,), jnp.float32)] * 2,
            [pltpu.VMEM((CHUNK,), jnp.float32)] * 2,
            pltpu.VMEM((LANES,), jnp.float32),
            pltpu.SemaphoreType.DMA,
            pltpu.SemaphoreType.DMA,
            pltpu.SemaphoreType.DMA,
            pltpu.SemaphoreType.DMA,
        ],
    )(_phys_view(z), _phys_view(l32), _phys_view(lp32), _phys_view(c32))

    tc_loss = pl.pallas_call(
        _tc_body,
        grid=(G0, C),
        in_specs=[
            pl.BlockSpec((1, HB, W), lambda g, c: (g // 8, g % 8, 0)),
            pl.BlockSpec((1, HB, W), lambda g, c: (g // 8, g % 8, 0)),
            pl.BlockSpec((1, HB, W), lambda g, c: (g // 8, g % 8, 0)),
            pl.BlockSpec((1, 1, HB, W), lambda g, c: (g // 8, c, g % 8, 0)),
        ],
        out_specs=pl.BlockSpec(memory_space=pltpu.SMEM),
        out_shape=jax.ShapeDtypeStruct((1, 1), jnp.float32),
        scratch_shapes=[
            pltpu.VMEM((HB, W), jnp.float32),
            pltpu.VMEM((HB, W), jnp.int32),
            pltpu.VMEM((HB, W), jnp.int32),
        ],
    )(l32, lp32, c32, z)

    loss = pl.pallas_call(
        _reduce_body,
        in_specs=[
            pl.BlockSpec((4, 128), lambda: (0, 0)),
            pl.BlockSpec(memory_space=pltpu.SMEM),
        ],
        out_shape=jax.ShapeDtypeStruct((1, 1), jnp.float32),
        out_specs=pl.BlockSpec(memory_space=pltpu.SMEM),
    )(partials.reshape(4, 128), tc_loss)
    return loss[0, 0]


# speedup vs baseline: 5.8220x; 5.7095x over previous
import jax
import jax.numpy as jnp
from jax import lax
from jax.experimental import pallas as pl
from jax.experimental.pallas import tpu as pltpu
from jax.experimental.pallas import tpu_sc as plsc

NC = 2
NS = 16
NW = NC * NS
LANES = 16

B, C, H, W = 4, 96, 512, 512
P = B * H * W
PW = P // NW
CHUNK = 4096
NCHUNK = PW // CHUNK
HW = H * W
LOG2_HW = 18


def _sc_body(z_hbm, l_hbm, lp_hbm, cond_hbm, out_hbm,
             l_v, lp_v, c_v, ig_v, ib_v, g_v, b_v, acc_v,
             sem_in0, sem_in1, sem_g0, sem_g1):
    wid = lax.axis_index("s") * NC + lax.axis_index("c")
    base = wid * PW
    iota = lax.iota(jnp.int32, LANES)
    sem_in = (sem_in0, sem_in1)
    sem_g = (sem_g0, sem_g1)

    def start_in(k):
        u = k % 2
        off = base + k * CHUNK
        return (
            pltpu.async_copy(l_hbm.at[pl.ds(off, CHUNK)], l_v[u], sem_in[u]),
            pltpu.async_copy(lp_hbm.at[pl.ds(off, CHUNK)], lp_v[u], sem_in[u]),
            pltpu.async_copy(cond_hbm.at[pl.ds(off, CHUNK)], c_v[u], sem_in[u]),
        )

    def do_idx(k):
        u = k % 2
        off = base + k * CHUNK

        @plsc.parallel_loop(0, CHUNK // LANES, unroll=8)
        def _(j):
            s = pl.ds(j * LANES, LANES)
            lv = l_v[u][s]
            lpv = lp_v[u][s]
            cv = c_v[u][s]
            p = (off + j * LANES) + iota
            bb = lax.shift_right_logical(p, LOG2_HW)
            pos = p + bb * ((C - 1) * HW)
            ig = pos + lax.shift_left(lv, LOG2_HW)
            ib = pos + lax.shift_left(lpv, LOG2_HW)
            ib = jnp.where(cv != 0, ib, ig)
            ig_v[u][s] = ig
            ib_v[u][s] = ib

    def start_g(k):
        u = k % 2
        return (
            pltpu.async_copy(z_hbm.at[ig_v[u]], g_v[u], sem_g[u]),
            pltpu.async_copy(z_hbm.at[ib_v[u]], b_v[u], sem_g[u]),
        )

    def do_acc(k, acc):
        u = k % 2

        @plsc.parallel_loop(0, CHUNK // LANES, unroll=8, carry=acc)
        def acc2(j, a):
            s = pl.ds(j * LANES, LANES)
            return a + (g_v[u][s] - b_v[u][s])

        return acc2

    acc = jnp.zeros((LANES,), jnp.float32)
    ins = [None] * (NCHUNK + 1)
    gs = [None] * (NCHUNK + 1)
    ins[0] = start_in(0)
    for d in ins[0]:
        d.wait()
    do_idx(0)
    ins[1] = start_in(1)
    gs[0] = start_g(0)
    for k in range(NCHUNK):
        if k + 1 < NCHUNK:
            for d in ins[k + 1]:
                d.wait()
            do_idx(k + 1)
            if k + 2 < NCHUNK:
                ins[k + 2] = start_in(k + 2)
            gs[k + 1] = start_g(k + 1)
        for d in gs[k]:
            d.wait()
        acc = do_acc(k, acc)

    acc_v[...] = acc
    pltpu.sync_copy(acc_v, out_hbm.at[wid])


def _reduce_body(x_ref, o_ref):
    o_ref[0, 0] = jnp.sum(x_ref[...])


def _phys_view(x):
    s = x.shape[:-2]
    n = len(s)
    x6 = x.reshape(*s, H // 8, 8, W // 128, 128)
    return jnp.transpose(x6, tuple(range(n)) + (n, n + 2, n + 1, n + 3)
                         ).reshape(-1)


@jax.jit
def kernel(z, condition, l, l_prime):
    z_flat = _phys_view(z)
    l_i = _phys_view(l.astype(jnp.int32))
    lp_i = _phys_view(l_prime.astype(jnp.int32))
    c_i = _phys_view(condition.astype(jnp.int32))

    mesh = plsc.VectorSubcoreMesh(
        core_axis_name="c", subcore_axis_name="s",
        num_cores=NC, num_subcores=NS)
    partials = pl.kernel(
        _sc_body,
        out_type=jax.ShapeDtypeStruct((NW, LANES), jnp.float32),
        mesh=mesh,
        scratch_types=[
            [pltpu.VMEM((CHUNK,), jnp.int32)] * 2,
            [pltpu.VMEM((CHUNK,), jnp.int32)] * 2,
            [pltpu.VMEM((CHUNK,), jnp.int32)] * 2,
            [pltpu.VMEM((CHUNK,), jnp.int32)] * 2,
            [pltpu.VMEM((CHUNK,), jnp.int32)] * 2,
            [pltpu.VMEM((CHUNK,), jnp.float32)] * 2,
            [pltpu.VMEM((CHUNK,), jnp.float32)] * 2,
            pltpu.VMEM((LANES,), jnp.float32),
            pltpu.SemaphoreType.DMA,
            pltpu.SemaphoreType.DMA,
            pltpu.SemaphoreType.DMA,
            pltpu.SemaphoreType.DMA,
        ],
    )(z_flat, l_i, lp_i, c_i)

    loss = pl.pallas_call(
        _reduce_body,
        out_shape=jax.ShapeDtypeStruct((1, 1), jnp.float32),
        out_specs=pl.BlockSpec(memory_space=pltpu.SMEM),
    )(partials.reshape(4, 128))
    return loss[0, 0]
